# baseline (speedup 1.0000x reference)
import functools

import jax
import jax.numpy as jnp
from jax import lax
from jax.experimental import pallas as pl
from jax.experimental.pallas import tpu as pltpu
from jax.experimental.pallas import tpu_sc as plsc

N = 10000
E = 320000
F = 128
H = 16
C = 40
C_PAD = 48

NC = 2
NS = 16
NW = NC * NS

CHUNK = 128
N_CHUNKS = 80
EPW = N_CHUNKS * CHUNK
E_PAD = NW * EPW
ROWS_PER_TILE = 640
N_PAD = NS * ROWS_PER_TILE


@functools.lru_cache(maxsize=None)
def _make_edge_agg(D):
    mesh = plsc.VectorSubcoreMesh(core_axis_name="c", subcore_axis_name="s")

    @functools.partial(
        pl.kernel,
        mesh=mesh,
        compiler_params=pltpu.CompilerParams(use_tc_tiling_on_sc=False),
        out_type=jax.ShapeDtypeStruct((NC, N_PAD, D), jnp.float32),
        scratch_types=[
            pltpu.VMEM((N_CHUNKS, CHUNK), jnp.int32),
            pltpu.VMEM((N_CHUNKS, CHUNK), jnp.int32),
            pltpu.VMEM((CHUNK, D), jnp.float32),
            pltpu.VMEM((CHUNK, D), jnp.float32),
            pltpu.VMEM_SHARED((N_PAD, D), jnp.float32),
            pltpu.SemaphoreType.DMA,
            pltpu.SemaphoreType.DMA,
        ],
    )
    def edge_agg(src_hbm, dst_hbm, t_hbm, out_hbm,
                 src_v, dst_v, rows_a, rows_b, acc, sem_a, sem_b):
        c = lax.axis_index("c")
        s = lax.axis_index("s")
        w = c * NS + s

        zvec = jnp.zeros((16,), jnp.float32)

        def zero_row(r, carry):
            for col in range(D // 16):
                rows_a[r, pl.ds(col * 16, 16)] = zvec
            return carry

        lax.fori_loop(0, CHUNK, zero_row, 0)
        for t in range(ROWS_PER_TILE // CHUNK):
            pltpu.sync_copy(rows_a, acc.at[pl.ds((s * 5 + t) * CHUNK, CHUNK)])

        pltpu.sync_copy(src_hbm.at[w], src_v)
        pltpu.sync_copy(dst_hbm.at[w], dst_v)

        plsc.subcore_barrier()

        def start_gather(j, buf, sem):
            pltpu.make_async_copy(t_hbm.at[src_v.at[j]], buf, sem).start()

        def wait_gather(j, buf, sem):
            pltpu.make_async_copy(t_hbm.at[src_v.at[j]], buf, sem).wait()

        start_gather(0, rows_a, sem_a)
        start_gather(1, rows_b, sem_b)

        def body(jj, carry):
            ja = 2 * jj
            jb = ja + 1
            wait_gather(ja, rows_a, sem_a)
            pltpu.sync_copy(rows_a, acc.at[dst_v.at[ja]], add=True)

            @pl.when(ja + 2 < N_CHUNKS)
            def _():
                start_gather(ja + 2, rows_a, sem_a)

            wait_gather(jb, rows_b, sem_b)
            pltpu.sync_copy(rows_b, acc.at[dst_v.at[jb]], add=True)

            @pl.when(jb + 2 < N_CHUNKS)
            def _():
                start_gather(jb + 2, rows_b, sem_b)

            return carry

        lax.fori_loop(0, N_CHUNKS // 2, body, 0)

        plsc.subcore_barrier()

        pltpu.sync_copy(acc.at[pl.ds(s * ROWS_PER_TILE, ROWS_PER_TILE)],
                        out_hbm.at[c, pl.ds(s * ROWS_PER_TILE, ROWS_PER_TILE)])

    return edge_agg


def _tc_in(x, W1):
    def body(x_ref, w_ref, o_ref):
        o_ref[...] = jnp.dot(x_ref[...], w_ref[...],
                             preferred_element_type=jnp.float32)

    return pl.pallas_call(
        body,
        grid=(5,),
        in_specs=[pl.BlockSpec((2000, F), lambda i: (i, 0)),
                  pl.BlockSpec((F, H), lambda i: (0, 0))],
        out_specs=pl.BlockSpec((2000, H), lambda i: (i, 0)),
        out_shape=jax.ShapeDtypeStruct((N, H), jnp.float32),
    )(x, W1)


def _tc_mid(p1, W2p, b1):
    def body(p_ref, w_ref, b_ref, o_ref):
        h = p_ref[0] + p_ref[1] + b_ref[...]
        h = jnp.maximum(h, 0.0)
        o_ref[...] = jnp.dot(h, w_ref[...], preferred_element_type=jnp.float32)

    return pl.pallas_call(
        body,
        grid=(10,),
        in_specs=[pl.BlockSpec((NC, 1024, H), lambda i: (0, i, 0)),
                  pl.BlockSpec((H, C_PAD), lambda i: (0, 0)),
                  pl.BlockSpec((1, H), lambda i: (0, 0))],
        out_specs=pl.BlockSpec((1024, C_PAD), lambda i: (i, 0)),
        out_shape=jax.ShapeDtypeStruct((N_PAD, C_PAD), jnp.float32),
    )(p1, W2p, b1.reshape(1, H))


def _tc_out(p2, b2):
    def body(p_ref, b_ref, o_ref):
        v = p_ref[0, :, :C] + p_ref[1, :, :C] + b_ref[...]
        m = jnp.max(v, axis=1, keepdims=True)
        e = jnp.exp(v - m)
        o_ref[...] = e / jnp.sum(e, axis=1, keepdims=True)

    return pl.pallas_call(
        body,
        grid=(10,),
        in_specs=[pl.BlockSpec((NC, 1000, C_PAD), lambda i: (0, i, 0)),
                  pl.BlockSpec((1, C), lambda i: (0, 0))],
        out_specs=pl.BlockSpec((1000, C), lambda i: (i, 0)),
        out_shape=jax.ShapeDtypeStruct((N, C), jnp.float32),
    )(p2, b2.reshape(1, C))


def kernel(x, edge_index, W1, b1, W2, b2):
    ei = edge_index.astype(jnp.int32)
    pad = E_PAD - E
    pad_dst = N + (jnp.arange(pad, dtype=jnp.int32) % (N_PAD - N))
    src = jnp.concatenate([ei[0], jnp.zeros((pad,), jnp.int32)])
    dst = jnp.concatenate([ei[1], pad_dst])
    src_r = src.reshape(NW, N_CHUNKS, CHUNK)
    dst_r = dst.reshape(NW, N_CHUNKS, CHUNK)
    W2p = jnp.pad(W2, ((0, 0), (0, C_PAD - C)))

    t1 = _tc_in(x, W1)
    p1 = _make_edge_agg(H)(src_r, dst_r, t1)
    t2 = _tc_mid(p1, W2p, b1)
    p2 = _make_edge_agg(C_PAD)(src_r, dst_r, t2)
    return _tc_out(p2, b2)


# speedup vs baseline: 1.0141x; 1.0141x over previous
import functools

import jax
import jax.numpy as jnp
from jax import lax
from jax.experimental import pallas as pl
from jax.experimental.pallas import tpu as pltpu
from jax.experimental.pallas import tpu_sc as plsc

N = 10000
E = 320000
F = 128
H = 16
C = 40
C_PAD = 48

NC = 2
NS = 16
NW = NC * NS

CHUNK = 128
N_CHUNKS = 80
DEPTH = 8
EPW = N_CHUNKS * CHUNK
E_PAD = NW * EPW
ROWS_PER_TILE = 640
N_PAD = NS * ROWS_PER_TILE


@functools.lru_cache(maxsize=None)
def _make_edge_agg(D):
    mesh = plsc.VectorSubcoreMesh(core_axis_name="c", subcore_axis_name="s")

    @functools.partial(
        pl.kernel,
        mesh=mesh,
        compiler_params=pltpu.CompilerParams(use_tc_tiling_on_sc=False),
        out_type=jax.ShapeDtypeStruct((NC, N_PAD, D), jnp.float32),
        scratch_types=[
            pltpu.VMEM((N_CHUNKS, CHUNK), jnp.int32),
            pltpu.VMEM((N_CHUNKS, CHUNK), jnp.int32),
            [pltpu.VMEM((CHUNK, D), jnp.float32) for _ in range(DEPTH)],
            [pltpu.SemaphoreType.DMA for _ in range(DEPTH)],
            pltpu.VMEM_SHARED((N_PAD, D), jnp.float32),
        ],
    )
    def edge_agg(src_hbm, dst_hbm, t_hbm, out_hbm,
                 src_v, dst_v, rows, sems, acc):
        c = lax.axis_index("c")
        s = lax.axis_index("s")
        w = c * NS + s

        zvec = jnp.zeros((16,), jnp.float32)

        def zero_row(r, carry):
            for col in range(D // 16):
                rows[0][r, pl.ds(col * 16, 16)] = zvec
            return carry

        lax.fori_loop(0, CHUNK, zero_row, 0)
        for t in range(ROWS_PER_TILE // CHUNK):
            pltpu.sync_copy(rows[0], acc.at[pl.ds((s * 5 + t) * CHUNK, CHUNK)])

        pltpu.sync_copy(src_hbm.at[w], src_v)
        pltpu.sync_copy(dst_hbm.at[w], dst_v)

        plsc.subcore_barrier()

        def start_gather(j, k):
            pltpu.make_async_copy(t_hbm.at[src_v.at[j]], rows[k], sems[k]).start()

        def wait_gather(j, k):
            pltpu.make_async_copy(t_hbm.at[src_v.at[j]], rows[k], sems[k]).wait()

        for k in range(DEPTH):
            start_gather(k, k)

        def body(jj, carry):
            for k in range(DEPTH):
                j = DEPTH * jj + k
                wait_gather(j, k)
                pltpu.sync_copy(rows[k], acc.at[dst_v.at[j]], add=True)

                @pl.when(j + DEPTH < N_CHUNKS)
                def _():
                    start_gather(j + DEPTH, k)

            return carry

        lax.fori_loop(0, N_CHUNKS // DEPTH, body, 0)

        plsc.subcore_barrier()

        pltpu.sync_copy(acc.at[pl.ds(s * ROWS_PER_TILE, ROWS_PER_TILE)],
                        out_hbm.at[c, pl.ds(s * ROWS_PER_TILE, ROWS_PER_TILE)])

    return edge_agg


def _tc_in(x, W1):
    def body(x_ref, w_ref, o_ref):
        o_ref[...] = jnp.dot(x_ref[...], w_ref[...],
                             preferred_element_type=jnp.float32)

    return pl.pallas_call(
        body,
        grid=(5,),
        in_specs=[pl.BlockSpec((2000, F), lambda i: (i, 0)),
                  pl.BlockSpec((F, H), lambda i: (0, 0))],
        out_specs=pl.BlockSpec((2000, H), lambda i: (i, 0)),
        out_shape=jax.ShapeDtypeStruct((N, H), jnp.float32),
    )(x, W1)


def _tc_mid(p1, W2p, b1):
    def body(p_ref, w_ref, b_ref, o_ref):
        h = p_ref[0] + p_ref[1] + b_ref[...]
        h = jnp.maximum(h, 0.0)
        o_ref[...] = jnp.dot(h, w_ref[...], preferred_element_type=jnp.float32)

    return pl.pallas_call(
        body,
        grid=(10,),
        in_specs=[pl.BlockSpec((NC, 1024, H), lambda i: (0, i, 0)),
                  pl.BlockSpec((H, C_PAD), lambda i: (0, 0)),
                  pl.BlockSpec((1, H), lambda i: (0, 0))],
        out_specs=pl.BlockSpec((1024, C_PAD), lambda i: (i, 0)),
        out_shape=jax.ShapeDtypeStruct((N_PAD, C_PAD), jnp.float32),
    )(p1, W2p, b1.reshape(1, H))


def _tc_out(p2, b2):
    def body(p_ref, b_ref, o_ref):
        v = p_ref[0, :, :C] + p_ref[1, :, :C] + b_ref[...]
        m = jnp.max(v, axis=1, keepdims=True)
        e = jnp.exp(v - m)
        o_ref[...] = e / jnp.sum(e, axis=1, keepdims=True)

    return pl.pallas_call(
        body,
        grid=(10,),
        in_specs=[pl.BlockSpec((NC, 1000, C_PAD), lambda i: (0, i, 0)),
                  pl.BlockSpec((1, C), lambda i: (0, 0))],
        out_specs=pl.BlockSpec((1000, C), lambda i: (i, 0)),
        out_shape=jax.ShapeDtypeStruct((N, C), jnp.float32),
    )(p2, b2.reshape(1, C))


def kernel(x, edge_index, W1, b1, W2, b2):
    ei = edge_index.astype(jnp.int32)
    pad = E_PAD - E
    pad_dst = N + (jnp.arange(pad, dtype=jnp.int32) % (N_PAD - N))
    src = jnp.concatenate([ei[0], jnp.zeros((pad,), jnp.int32)])
    dst = jnp.concatenate([ei[1], pad_dst])
    dst_r = dst.reshape(NW, N_CHUNKS, CHUNK)
    W2p = jnp.pad(W2, ((0, 0), (0, C_PAD - C)))

    src_r = src.reshape(NW, N_CHUNKS, CHUNK)
    t1 = _tc_in(x, W1)
    p1 = _make_edge_agg(H)(src_r, dst_r, t1)
    t2 = _tc_mid(p1, W2p, b1)
    p2 = _make_edge_agg(C_PAD)(src_r, dst_r, t2)
    return _tc_out(p2, b2)


# speedup vs baseline: 1.8561x; 1.8302x over previous
import functools

import jax
import jax.numpy as jnp
from jax import lax
from jax.experimental import pallas as pl
from jax.experimental.pallas import tpu as pltpu
from jax.experimental.pallas import tpu_sc as plsc

N = 10000
E = 320000
F = 128
H = 16
C = 40
C_PAD = 48

NC = 2
NS = 16
NW = NC * NS

CHUNK = 128
N_CHUNKS = 80
DEPTH = 4
TROWS = N // NS
EPW = N_CHUNKS * CHUNK
E_PAD = NW * EPW
ROWS_PER_TILE = 640
N_PAD = NS * ROWS_PER_TILE


@functools.lru_cache(maxsize=None)
def _make_edge_agg(D):
    mesh = plsc.VectorSubcoreMesh(core_axis_name="c", subcore_axis_name="s")

    @functools.partial(
        pl.kernel,
        mesh=mesh,
        compiler_params=pltpu.CompilerParams(use_tc_tiling_on_sc=False),
        out_type=jax.ShapeDtypeStruct((NC, N_PAD, D), jnp.float32),
        scratch_types=[
            pltpu.VMEM((N_CHUNKS, CHUNK), jnp.int32),
            pltpu.VMEM((N_CHUNKS, CHUNK), jnp.int32),
            [pltpu.VMEM((CHUNK, D), jnp.float32) for _ in range(DEPTH)],
            [pltpu.SemaphoreType.DMA for _ in range(DEPTH)],
            pltpu.VMEM_SHARED((N_PAD, D), jnp.float32),
            pltpu.VMEM_SHARED((N, D), jnp.float32),
        ],
    )
    def edge_agg(src_hbm, dst_hbm, t_hbm, out_hbm,
                 src_v, dst_v, rows, sems, acc, tbl):
        c = lax.axis_index("c")
        s = lax.axis_index("s")
        w = c * NS + s

        zvec = jnp.zeros((16,), jnp.float32)

        def zero_row(r, carry):
            for col in range(D // 16):
                rows[0][r, pl.ds(col * 16, 16)] = zvec
            return carry

        lax.fori_loop(0, CHUNK, zero_row, 0)
        for t in range(ROWS_PER_TILE // CHUNK):
            pltpu.sync_copy(rows[0], acc.at[pl.ds((s * 5 + t) * CHUNK, CHUNK)])

        pltpu.sync_copy(t_hbm.at[pl.ds(s * TROWS, TROWS)],
                        tbl.at[pl.ds(s * TROWS, TROWS)])
        pltpu.sync_copy(src_hbm.at[w], src_v)
        pltpu.sync_copy(dst_hbm.at[w], dst_v)

        plsc.subcore_barrier()

        def start_gather(j, k):
            pltpu.make_async_copy(tbl.at[src_v.at[j]], rows[k], sems[k]).start()

        def wait_gather(j, k):
            pltpu.make_async_copy(tbl.at[src_v.at[j]], rows[k], sems[k]).wait()

        for k in range(DEPTH):
            start_gather(k, k)

        def body(jj, carry):
            for k in range(DEPTH):
                j = DEPTH * jj + k
                wait_gather(j, k)
                pltpu.sync_copy(rows[k], acc.at[dst_v.at[j]], add=True)

                @pl.when(j + DEPTH < N_CHUNKS)
                def _():
                    start_gather(j + DEPTH, k)

            return carry

        lax.fori_loop(0, N_CHUNKS // DEPTH, body, 0)

        plsc.subcore_barrier()

        pltpu.sync_copy(acc.at[pl.ds(s * ROWS_PER_TILE, ROWS_PER_TILE)],
                        out_hbm.at[c, pl.ds(s * ROWS_PER_TILE, ROWS_PER_TILE)])

    return edge_agg


def _tc_in(x, W1):
    def body(x_ref, w_ref, o_ref):
        o_ref[...] = jnp.dot(x_ref[...], w_ref[...],
                             preferred_element_type=jnp.float32)

    return pl.pallas_call(
        body,
        grid=(5,),
        in_specs=[pl.BlockSpec((2000, F), lambda i: (i, 0)),
                  pl.BlockSpec((F, H), lambda i: (0, 0))],
        out_specs=pl.BlockSpec((2000, H), lambda i: (i, 0)),
        out_shape=jax.ShapeDtypeStruct((N, H), jnp.float32),
    )(x, W1)


def _tc_mid(p1, W2p, b1):
    def body(p_ref, w_ref, b_ref, o_ref):
        h = p_ref[0] + p_ref[1] + b_ref[...]
        h = jnp.maximum(h, 0.0)
        o_ref[...] = jnp.dot(h, w_ref[...], preferred_element_type=jnp.float32)

    return pl.pallas_call(
        body,
        grid=(10,),
        in_specs=[pl.BlockSpec((NC, 1024, H), lambda i: (0, i, 0)),
                  pl.BlockSpec((H, C_PAD), lambda i: (0, 0)),
                  pl.BlockSpec((1, H), lambda i: (0, 0))],
        out_specs=pl.BlockSpec((1024, C_PAD), lambda i: (i, 0)),
        out_shape=jax.ShapeDtypeStruct((N_PAD, C_PAD), jnp.float32),
    )(p1, W2p, b1.reshape(1, H))


def _tc_out(p2, b2):
    def body(p_ref, b_ref, o_ref):
        v = p_ref[0, :, :C] + p_ref[1, :, :C] + b_ref[...]
        m = jnp.max(v, axis=1, keepdims=True)
        e = jnp.exp(v - m)
        o_ref[...] = e / jnp.sum(e, axis=1, keepdims=True)

    return pl.pallas_call(
        body,
        grid=(10,),
        in_specs=[pl.BlockSpec((NC, 1000, C_PAD), lambda i: (0, i, 0)),
                  pl.BlockSpec((1, C), lambda i: (0, 0))],
        out_specs=pl.BlockSpec((1000, C), lambda i: (i, 0)),
        out_shape=jax.ShapeDtypeStruct((N, C), jnp.float32),
    )(p2, b2.reshape(1, C))


def kernel(x, edge_index, W1, b1, W2, b2):
    ei = edge_index.astype(jnp.int32)
    pad = E_PAD - E
    pad_dst = N + (jnp.arange(pad, dtype=jnp.int32) % (N_PAD - N))
    src = jnp.concatenate([ei[0], jnp.zeros((pad,), jnp.int32)])
    dst = jnp.concatenate([ei[1], pad_dst])
    dst_r = dst.reshape(NW, N_CHUNKS, CHUNK)
    W2p = jnp.pad(W2, ((0, 0), (0, C_PAD - C)))

    src_r = src.reshape(NW, N_CHUNKS, CHUNK)
    t1 = _tc_in(x, W1)
    p1 = _make_edge_agg(H)(src_r, dst_r, t1)
    t2 = _tc_mid(p1, W2p, b1)
    p2 = _make_edge_agg(C_PAD)(src_r, dst_r, t2)
    return _tc_out(p2, b2)


# speedup vs baseline: 2.0701x; 1.1153x over previous
import functools

import jax
import jax.numpy as jnp
from jax import lax
from jax.experimental import pallas as pl
from jax.experimental.pallas import tpu as pltpu
from jax.experimental.pallas import tpu_sc as plsc

N = 10000
E = 320000
F = 128
H = 16
C = 40
C_PAD = 48

NC = 2
NS = 16
NW = NC * NS

EPT = E // NW
CHUNK = 128
N_FULL = EPT // CHUNK
TAIL = EPT - N_FULL * CHUNK
DEPTH = 6
ROWS_PER_TILE = 640
N_PAD = NS * ROWS_PER_TILE
TROWS = N // NS


@functools.lru_cache(maxsize=None)
def _make_edge_agg(D):
    mesh = plsc.VectorSubcoreMesh(core_axis_name="c", subcore_axis_name="s")

    @functools.partial(
        pl.kernel,
        mesh=mesh,
        compiler_params=pltpu.CompilerParams(use_tc_tiling_on_sc=False),
        out_type=jax.ShapeDtypeStruct((NC, N_PAD, D), jnp.float32),
        scratch_types=[
            pltpu.VMEM((EPT,), jnp.int32),
            pltpu.VMEM((EPT,), jnp.int32),
            [pltpu.VMEM((CHUNK, D), jnp.float32) for _ in range(DEPTH)],
            [pltpu.SemaphoreType.DMA for _ in range(DEPTH)],
            pltpu.VMEM_SHARED((N_PAD, D), jnp.float32),
            pltpu.VMEM_SHARED((N, D), jnp.float32),
        ],
    )
    def edge_agg(ei_hbm, t_hbm, out_hbm, src_v, dst_v, rows, sems, acc, tbl):
        c = lax.axis_index("c")
        s = lax.axis_index("s")
        w = c * NS + s

        zvec = jnp.zeros((16,), jnp.float32)

        def zero_row(r, carry):
            for col in range(D // 16):
                rows[0][r, pl.ds(col * 16, 16)] = zvec
            return carry

        lax.fori_loop(0, CHUNK, zero_row, 0)
        for t in range(ROWS_PER_TILE // CHUNK):
            pltpu.sync_copy(rows[0], acc.at[pl.ds((s * 5 + t) * CHUNK, CHUNK)])

        pltpu.sync_copy(t_hbm.at[pl.ds(s * TROWS, TROWS)],
                        tbl.at[pl.ds(s * TROWS, TROWS)])
        pltpu.sync_copy(ei_hbm.at[0, pl.ds(w * EPT, EPT)], src_v)
        pltpu.sync_copy(ei_hbm.at[1, pl.ds(w * EPT, EPT)], dst_v)

        plsc.subcore_barrier()

        def start_gather(j, k):
            pltpu.make_async_copy(tbl.at[src_v.at[pl.ds(j * CHUNK, CHUNK)]],
                                  rows[k], sems[k]).start()

        def wait_gather(j, k):
            pltpu.make_async_copy(tbl.at[src_v.at[pl.ds(j * CHUNK, CHUNK)]],
                                  rows[k], sems[k]).wait()

        for k in range(DEPTH):
            start_gather(k, k)

        def body(jj, carry):
            for k in range(DEPTH):
                j = DEPTH * jj + k
                wait_gather(j, k)
                pltpu.sync_copy(rows[k],
                                acc.at[dst_v.at[pl.ds(j * CHUNK, CHUNK)]],
                                add=True)

                @pl.when(j + DEPTH < N_FULL)
                def _():
                    start_gather(j + DEPTH, k)

            return carry

        lax.fori_loop(0, N_FULL // DEPTH, body, 0)

        tail = N_FULL * CHUNK
        pltpu.make_async_copy(tbl.at[src_v.at[pl.ds(tail, TAIL)]],
                              rows[0].at[pl.ds(0, TAIL)], sems[0]).start()
        pltpu.make_async_copy(tbl.at[src_v.at[pl.ds(tail, TAIL)]],
                              rows[0].at[pl.ds(0, TAIL)], sems[0]).wait()
        pltpu.sync_copy(rows[0].at[pl.ds(0, TAIL)],
                        acc.at[dst_v.at[pl.ds(tail, TAIL)]], add=True)

        plsc.subcore_barrier()

        pltpu.sync_copy(acc.at[pl.ds(s * ROWS_PER_TILE, ROWS_PER_TILE)],
                        out_hbm.at[c, pl.ds(s * ROWS_PER_TILE, ROWS_PER_TILE)])

    return edge_agg


def _tc_in(x, W1):
    def body(x_ref, w_ref, o_ref):
        o_ref[...] = jnp.dot(x_ref[...], w_ref[...],
                             preferred_element_type=jnp.float32)

    return pl.pallas_call(
        body,
        out_shape=jax.ShapeDtypeStruct((N, H), jnp.float32),
    )(x, W1)


def _tc_mid(p1, W2p, b1):
    def body(p_ref, w_ref, b_ref, o_ref):
        h = p_ref[0] + p_ref[1] + b_ref[...]
        h = jnp.maximum(h, 0.0)
        o_ref[...] = jnp.dot(h, w_ref[...], preferred_element_type=jnp.float32)

    return pl.pallas_call(
        body,
        out_shape=jax.ShapeDtypeStruct((N_PAD, C_PAD), jnp.float32),
    )(p1, W2p, b1.reshape(1, H))


def _tc_out(p2, b2):
    def body(p_ref, b_ref, o_ref):
        v = p_ref[0, :N, :C] + p_ref[1, :N, :C] + b_ref[...]
        m = jnp.max(v, axis=1, keepdims=True)
        e = jnp.exp(v - m)
        o_ref[...] = e / jnp.sum(e, axis=1, keepdims=True)

    return pl.pallas_call(
        body,
        out_shape=jax.ShapeDtypeStruct((N, C), jnp.float32),
    )(p2, b2.reshape(1, C))


def kernel(x, edge_index, W1, b1, W2, b2):
    ei = edge_index.astype(jnp.int32)
    W2p = jnp.pad(W2, ((0, 0), (0, C_PAD - C)))

    t1 = _tc_in(x, W1)
    p1 = _make_edge_agg(H)(ei, t1)
    t2 = _tc_mid(p1, W2p, b1)
    p2 = _make_edge_agg(C_PAD)(ei, t2)
    return _tc_out(p2, b2)


# speedup vs baseline: 2.1388x; 1.0332x over previous
import functools

import jax
import jax.numpy as jnp
from jax import lax
from jax.experimental import pallas as pl
from jax.experimental.pallas import tpu as pltpu
from jax.experimental.pallas import tpu_sc as plsc

N = 10000
E = 320000
F = 128
H = 16
C = 40

NC = 2
NS = 16
NW = NC * NS

EPT = E // NW
CHUNK = 128
N_FULL = EPT // CHUNK
TAIL = EPT - N_FULL * CHUNK
DEPTH = 6
ROWS_PER_TILE = 640
N_PAD = NS * ROWS_PER_TILE
TROWS = N // NS


@functools.lru_cache(maxsize=None)
def _make_edge_agg(D):
    mesh = plsc.VectorSubcoreMesh(core_axis_name="c", subcore_axis_name="s")

    @functools.partial(
        pl.kernel,
        mesh=mesh,
        compiler_params=pltpu.CompilerParams(use_tc_tiling_on_sc=False),
        out_type=jax.ShapeDtypeStruct((NC, N_PAD, D), jnp.float32),
        scratch_types=[
            pltpu.VMEM((EPT,), jnp.int32),
            pltpu.VMEM((EPT,), jnp.int32),
            [pltpu.VMEM((CHUNK, D), jnp.float32) for _ in range(DEPTH)],
            [pltpu.SemaphoreType.DMA for _ in range(DEPTH)],
            pltpu.VMEM_SHARED((N_PAD, D), jnp.float32),
            pltpu.VMEM_SHARED((N, D), jnp.float32),
        ],
    )
    def edge_agg(ei_hbm, t_hbm, z_hbm, out_hbm, src_v, dst_v, rows, sems, acc, tbl):
        c = lax.axis_index("c")
        s = lax.axis_index("s")
        w = c * NS + s

        pltpu.sync_copy(z_hbm, acc.at[pl.ds(s * ROWS_PER_TILE, ROWS_PER_TILE)])

        pltpu.sync_copy(t_hbm.at[pl.ds(s * TROWS, TROWS)],
                        tbl.at[pl.ds(s * TROWS, TROWS)])
        pltpu.sync_copy(ei_hbm.at[0, pl.ds(w * EPT, EPT)], src_v)
        pltpu.sync_copy(ei_hbm.at[1, pl.ds(w * EPT, EPT)], dst_v)

        plsc.subcore_barrier()

        def start_gather(j, k):
            pltpu.make_async_copy(tbl.at[src_v.at[pl.ds(j * CHUNK, CHUNK)]],
                                  rows[k], sems[k]).start()

        def wait_gather(j, k):
            pltpu.make_async_copy(tbl.at[src_v.at[pl.ds(j * CHUNK, CHUNK)]],
                                  rows[k], sems[k]).wait()

        for k in range(DEPTH):
            start_gather(k, k)

        def body(jj, carry):
            for k in range(DEPTH):
                j = DEPTH * jj + k
                wait_gather(j, k)
                pltpu.sync_copy(rows[k],
                                acc.at[dst_v.at[pl.ds(j * CHUNK, CHUNK)]],
                                add=True)

                @pl.when(j + DEPTH < N_FULL)
                def _():
                    start_gather(j + DEPTH, k)

            return carry

        lax.fori_loop(0, N_FULL // DEPTH, body, 0)

        tail = N_FULL * CHUNK
        pltpu.make_async_copy(tbl.at[src_v.at[pl.ds(tail, TAIL)]],
                              rows[0].at[pl.ds(0, TAIL)], sems[0]).start()
        pltpu.make_async_copy(tbl.at[src_v.at[pl.ds(tail, TAIL)]],
                              rows[0].at[pl.ds(0, TAIL)], sems[0]).wait()
        pltpu.sync_copy(rows[0].at[pl.ds(0, TAIL)],
                        acc.at[dst_v.at[pl.ds(tail, TAIL)]], add=True)

        plsc.subcore_barrier()

        pltpu.sync_copy(acc.at[pl.ds(s * ROWS_PER_TILE, ROWS_PER_TILE)],
                        out_hbm.at[c, pl.ds(s * ROWS_PER_TILE, ROWS_PER_TILE)])

    return edge_agg


def _tc_in(x, W1):
    def body(x_ref, w_ref, o_ref):
        o_ref[...] = jnp.dot(x_ref[...], w_ref[...],
                             preferred_element_type=jnp.float32)

    return pl.pallas_call(
        body,
        out_shape=jax.ShapeDtypeStruct((N, H), jnp.float32),
    )(x, W1)


def _tc_mid(p1, W2, b1):
    def body(p_ref, w_ref, b_ref, o_ref):
        h = p_ref[0] + p_ref[1] + b_ref[...]
        h = jnp.maximum(h, 0.0)
        o_ref[...] = jnp.dot(h, w_ref[...], preferred_element_type=jnp.float32)

    return pl.pallas_call(
        body,
        out_shape=jax.ShapeDtypeStruct((N_PAD, C), jnp.float32),
    )(p1, W2, b1.reshape(1, H))


def _tc_out(p2, b2):
    def body(p_ref, b_ref, o_ref):
        v = p_ref[0, :N, :] + p_ref[1, :N, :] + b_ref[...]
        m = jnp.max(v, axis=1, keepdims=True)
        e = jnp.exp(v - m)
        o_ref[...] = e / jnp.sum(e, axis=1, keepdims=True)

    return pl.pallas_call(
        body,
        out_shape=jax.ShapeDtypeStruct((N, C), jnp.float32),
    )(p2, b2.reshape(1, C))


def kernel(x, edge_index, W1, b1, W2, b2):
    ei = edge_index.astype(jnp.int32)
    z1 = jnp.zeros((ROWS_PER_TILE, H), jnp.float32)
    z2 = jnp.zeros((ROWS_PER_TILE, C), jnp.float32)

    t1 = _tc_in(x, W1)
    p1 = _make_edge_agg(H)(ei, t1, z1)
    t2 = _tc_mid(p1, W2, b1)
    p2 = _make_edge_agg(C)(ei, t2, z2)
    return _tc_out(p2, b2)
